# baseline (speedup 1.0000x reference)
import functools

import jax
import jax.numpy as jnp
from jax import lax
from jax.experimental import pallas as pl
from jax.experimental.pallas import tpu as pltpu
from jax.experimental.pallas import tpu_sc as plsc


S, D, E, K = 2048, 768, 8, 2
N = S * K
BLK = 256
NPAD = N + E * BLK
NB = NPAD // BLK
CB = 512


def _gate_kernel(x_ref, wgt_ref, bg_ref, meta_ref):
    xb = x_ref[...]
    logits = jnp.dot(xb, wgt_ref[...], preferred_element_type=jnp.float32)
    logits = logits + bg_ref[...]

    iota = lax.broadcasted_iota(jnp.int32, (S, E), 1)
    i1 = jnp.argmax(logits, axis=-1)[:, None]
    one1 = iota == i1
    v1 = jnp.max(logits, axis=-1, keepdims=True)
    masked = jnp.where(one1, -jnp.inf, logits)
    i2 = jnp.argmax(masked, axis=-1)[:, None]
    v2 = jnp.max(masked, axis=-1, keepdims=True)

    t = jnp.exp(v2 - v1)
    denom = 1.0 + t
    p1 = 1.0 / denom
    p2 = t / denom

    i1f = i1.astype(jnp.float32)
    i2f = i2.astype(jnp.float32)
    meta = jnp.where(iota == 0, i1f,
           jnp.where(iota == 1, i2f,
           jnp.where(iota == 2, p1,
           jnp.where(iota == 3, p2, 0.0))))
    meta_ref[...] = meta


def _gate(x2, WgT, bg2):
    return pl.pallas_call(
        _gate_kernel,
        grid=(1,),
        in_specs=[
            pl.BlockSpec((S, D), lambda i: (0, 0)),
            pl.BlockSpec((D, E), lambda i: (0, 0)),
            pl.BlockSpec((1, E), lambda i: (0, 0)),
        ],
        out_specs=pl.BlockSpec((S, E), lambda i: (0, 0)),
        out_shape=jax.ShapeDtypeStruct((S, E), jnp.float32),
    )(x2, WgT, bg2)


def _route(meta):
    i1 = meta[:, 0].astype(jnp.int32)
    i2 = meta[:, 1].astype(jnp.int32)
    eid = jnp.concatenate([i1, i2])
    oh = (eid[:, None] == jnp.arange(E)[None, :]).astype(jnp.int32)
    ranks_inc = jnp.cumsum(oh, axis=0)
    rank = jnp.take_along_axis(ranks_inc, eid[:, None], axis=1)[:, 0] - 1
    counts = ranks_inc[-1]
    padded = ((counts + BLK - 1) // BLK) * BLK
    seg_end = jnp.cumsum(padded)
    seg_start = seg_end - padded
    dst = seg_start[eid] + rank
    g_tok = jnp.zeros((NPAD,), jnp.int32).at[dst].set(
        jnp.arange(N, dtype=jnp.int32) % S)
    block_first = jnp.arange(NB, dtype=jnp.int32) * BLK
    blk_eid = jnp.sum((block_first[:, None] >= seg_end[None, :]).astype(jnp.int32),
                      axis=1)
    blk_eid = jnp.minimum(blk_eid, E - 1).astype(jnp.int32)
    return dst.astype(jnp.int32), g_tok, blk_eid


def _sc_gather(table, idx, n_rows):
    info = plsc.get_sparse_core_info()
    nw = info.num_cores * info.num_subcores
    b_per_w = n_rows // nw
    mesh = plsc.VectorSubcoreMesh(core_axis_name="c", subcore_axis_name="s")

    CH = 64

    @functools.partial(
        pl.kernel, mesh=mesh,
        out_type=jax.ShapeDtypeStruct((n_rows, D), jnp.float32),
        scratch_types=[
            pltpu.VMEM((b_per_w,), jnp.int32),
            pltpu.VMEM((CH, D), jnp.float32),
            pltpu.SemaphoreType.DMA,
        ],
    )
    def k(table_hbm, idx_hbm, out_hbm, idx_v, rows_v, sem):
        wid = lax.axis_index("s") * info.num_cores + lax.axis_index("c")
        base = wid * b_per_w
        pltpu.sync_copy(idx_hbm.at[pl.ds(base, b_per_w)], idx_v)

        @pl.loop(0, b_per_w, step=CH)
        def _(c):
            pltpu.async_copy(table_hbm.at[idx_v.at[pl.ds(c, CH)]], rows_v, sem).wait()
            pltpu.sync_copy(rows_v, out_hbm.at[pl.ds(base + c, CH)])

    return k(table, idx)


def _gmm_kernel(eid_ref, xg_ref, wt_ref, b_ref, y_ref):
    xg = xg_ref[...].astype(jnp.bfloat16)
    y = jnp.dot(xg, wt_ref[0], preferred_element_type=jnp.float32)
    y_ref[...] = y + b_ref[0]


def _gmm(blk_eid, xg, Wt, b):
    grid_spec = pltpu.PrefetchScalarGridSpec(
        num_scalar_prefetch=1,
        grid=(NB,),
        in_specs=[
            pl.BlockSpec((BLK, D), lambda m, eid: (m, 0)),
            pl.BlockSpec((1, D, D), lambda m, eid: (eid[m], 0, 0)),
            pl.BlockSpec((1, 1, D), lambda m, eid: (eid[m], 0, 0)),
        ],
        out_specs=pl.BlockSpec((BLK, D), lambda m, eid: (m, 0)),
    )
    return pl.pallas_call(
        _gmm_kernel,
        grid_spec=grid_spec,
        out_shape=jax.ShapeDtypeStruct((NPAD, D), jnp.float32),
    )(blk_eid, xg, Wt, b.reshape(E, 1, D))


def _combine_kernel(y1_ref, y2_ref, meta_ref, o_ref):
    p1 = meta_ref[:, 2:3]
    p2 = meta_ref[:, 3:4]
    o_ref[...] = p1 * y1_ref[...] + p2 * y2_ref[...]


def _combine(ycomb, meta):
    return pl.pallas_call(
        _combine_kernel,
        grid=(S // CB,),
        in_specs=[
            pl.BlockSpec((CB, D), lambda i: (i, 0)),
            pl.BlockSpec((CB, D), lambda i: (i + S // CB, 0)),
            pl.BlockSpec((CB, E), lambda i: (i, 0)),
        ],
        out_specs=pl.BlockSpec((CB, D), lambda i: (i, 0)),
        out_shape=jax.ShapeDtypeStruct((S, D), jnp.float32),
    )(ycomb, ycomb, meta)


@jax.jit
def kernel(x, Wg, bg, W, b):
    x2 = x.reshape(S, D)
    WgT = Wg.T
    bg2 = bg.reshape(1, E)
    Wt = W.transpose(0, 2, 1).astype(jnp.bfloat16)

    meta = _gate(x2, WgT, bg2)
    dst, g_tok, blk_eid = _route(meta)
    xg = _sc_gather(x2, g_tok, NPAD)
    y = _gmm(blk_eid, xg, Wt, b)
    ycomb = _sc_gather(y, dst, N)
    out = _combine(ycomb, meta)
    return out.reshape(1, S, D)


# speedup vs baseline: 1.7512x; 1.7512x over previous
import functools

import jax
import jax.numpy as jnp
from jax import lax
from jax.experimental import pallas as pl
from jax.experimental.pallas import tpu as pltpu
from jax.experimental.pallas import tpu_sc as plsc


S, D, E, K = 2048, 768, 8, 2
N = S * K
BLK = 256
NPAD = N + E * BLK
NB = NPAD // BLK
CB = 512


def _gate_kernel(x_ref, wgt_ref, bg_ref, meta_ref):
    xb = x_ref[...]
    logits = jnp.dot(xb, wgt_ref[...], preferred_element_type=jnp.float32)
    logits = logits + bg_ref[...]

    iota = lax.broadcasted_iota(jnp.int32, (S, E), 1)
    i1 = jnp.argmax(logits, axis=-1)[:, None]
    one1 = iota == i1
    v1 = jnp.max(logits, axis=-1, keepdims=True)
    masked = jnp.where(one1, -jnp.inf, logits)
    i2 = jnp.argmax(masked, axis=-1)[:, None]
    v2 = jnp.max(masked, axis=-1, keepdims=True)

    t = jnp.exp(v2 - v1)
    denom = 1.0 + t
    p1 = 1.0 / denom
    p2 = t / denom

    i1f = i1.astype(jnp.float32)
    i2f = i2.astype(jnp.float32)
    meta = jnp.where(iota == 0, i1f,
           jnp.where(iota == 1, i2f,
           jnp.where(iota == 2, p1,
           jnp.where(iota == 3, p2, 0.0))))
    meta_ref[...] = meta


def _gate(x2, WgT, bg2):
    return pl.pallas_call(
        _gate_kernel,
        grid=(1,),
        in_specs=[
            pl.BlockSpec((S, D), lambda i: (0, 0)),
            pl.BlockSpec((D, E), lambda i: (0, 0)),
            pl.BlockSpec((1, E), lambda i: (0, 0)),
        ],
        out_specs=pl.BlockSpec((S, E), lambda i: (0, 0)),
        out_shape=jax.ShapeDtypeStruct((S, E), jnp.float32),
    )(x2, WgT, bg2)


def _route(meta):
    i1 = meta[:, 0].astype(jnp.int32)
    i2 = meta[:, 1].astype(jnp.int32)
    eid = jnp.concatenate([i1, i2])
    oh = (eid[:, None] == jnp.arange(E)[None, :]).astype(jnp.int32)
    ranks_inc = jnp.cumsum(oh, axis=0)
    rank = jnp.take_along_axis(ranks_inc, eid[:, None], axis=1)[:, 0] - 1
    counts = ranks_inc[-1]
    padded = ((counts + BLK - 1) // BLK) * BLK
    seg_end = jnp.cumsum(padded)
    seg_start = seg_end - padded
    dst = seg_start[eid] + rank
    g_tok = (jnp.arange(NPAD, dtype=jnp.int32) % S).at[dst].set(
        jnp.arange(N, dtype=jnp.int32) % S)
    block_first = jnp.arange(NB, dtype=jnp.int32) * BLK
    blk_eid = jnp.sum((block_first[:, None] >= seg_end[None, :]).astype(jnp.int32),
                      axis=1)
    blk_eid = jnp.minimum(blk_eid, E - 1).astype(jnp.int32)
    return dst.astype(jnp.int32), g_tok, blk_eid


def _sc_gather(table, idx, n_rows):
    info = plsc.get_sparse_core_info()
    nw = info.num_cores * info.num_subcores
    b_per_w = n_rows // nw
    mesh = plsc.VectorSubcoreMesh(core_axis_name="c", subcore_axis_name="s")

    CH = 64

    @functools.partial(
        pl.kernel, mesh=mesh,
        out_type=jax.ShapeDtypeStruct((n_rows, D), jnp.float32),
        scratch_types=[
            pltpu.VMEM((b_per_w,), jnp.int32),
            pltpu.VMEM((CH, D), jnp.float32),
            pltpu.SemaphoreType.DMA,
        ],
    )
    def k(table_hbm, idx_hbm, out_hbm, idx_v, rows_v, sem):
        wid = lax.axis_index("s") * info.num_cores + lax.axis_index("c")
        base = wid * b_per_w
        pltpu.sync_copy(idx_hbm.at[pl.ds(base, b_per_w)], idx_v)

        @pl.loop(0, b_per_w, step=CH)
        def _(c):
            pltpu.async_copy(table_hbm.at[idx_v.at[pl.ds(c, CH)]], rows_v, sem).wait()
            pltpu.sync_copy(rows_v, out_hbm.at[pl.ds(base + c, CH)])

    return k(table, idx)


def _gmm_kernel(eid_ref, xg_ref, wt_ref, b_ref, y_ref):
    xg = xg_ref[...].astype(jnp.bfloat16)
    y = jnp.dot(xg, wt_ref[0], preferred_element_type=jnp.float32)
    y_ref[...] = y + b_ref[0]


def _gmm(blk_eid, xg, Wt, b):
    grid_spec = pltpu.PrefetchScalarGridSpec(
        num_scalar_prefetch=1,
        grid=(NB,),
        in_specs=[
            pl.BlockSpec((BLK, D), lambda m, eid: (m, 0)),
            pl.BlockSpec((1, D, D), lambda m, eid: (eid[m], 0, 0)),
            pl.BlockSpec((1, 1, D), lambda m, eid: (eid[m], 0, 0)),
        ],
        out_specs=pl.BlockSpec((BLK, D), lambda m, eid: (m, 0)),
    )
    return pl.pallas_call(
        _gmm_kernel,
        grid_spec=grid_spec,
        out_shape=jax.ShapeDtypeStruct((NPAD, D), jnp.float32),
    )(blk_eid, xg, Wt, b.reshape(E, 1, D))


def _combine_kernel(y1_ref, y2_ref, meta_ref, o_ref):
    p1 = meta_ref[:, 2:3]
    p2 = meta_ref[:, 3:4]
    o_ref[...] = p1 * y1_ref[...] + p2 * y2_ref[...]


def _combine(ycomb, meta):
    return pl.pallas_call(
        _combine_kernel,
        grid=(S // CB,),
        in_specs=[
            pl.BlockSpec((CB, D), lambda i: (i, 0)),
            pl.BlockSpec((CB, D), lambda i: (i + S // CB, 0)),
            pl.BlockSpec((CB, E), lambda i: (i, 0)),
        ],
        out_specs=pl.BlockSpec((CB, D), lambda i: (i, 0)),
        out_shape=jax.ShapeDtypeStruct((S, D), jnp.float32),
    )(ycomb, ycomb, meta)


@jax.jit
def kernel(x, Wg, bg, W, b):
    x2 = x.reshape(S, D)
    WgT = Wg.T
    bg2 = bg.reshape(1, E)
    Wt = W.transpose(0, 2, 1).astype(jnp.bfloat16)

    meta = _gate(x2, WgT, bg2)
    dst, g_tok, blk_eid = _route(meta)
    xg = _sc_gather(x2, g_tok, NPAD)
    y = _gmm(blk_eid, xg, Wt, b)
    ycomb = _sc_gather(y, dst, N)
    out = _combine(ycomb, meta)
    return out.reshape(1, S, D)


# speedup vs baseline: 5.7156x; 3.2639x over previous
import jax
import jax.numpy as jnp
from jax.experimental import pallas as pl
from jax.experimental.pallas import tpu as pltpu


S, D, E = 2048, 768, 8
BS = 512


def _moe_block(x_ref, wgt_ref, bg_ref, wt_ref, b_ref, o_ref):
    xb = x_ref[...]
    logits = jnp.dot(xb, wgt_ref[...], preferred_element_type=jnp.float32)
    logits = logits + bg_ref[...]

    iota = jax.lax.broadcasted_iota(jnp.int32, (BS, E), 1)
    i1 = jnp.argmax(logits, axis=-1)[:, None]
    one1 = iota == i1
    v1 = jnp.max(logits, axis=-1, keepdims=True)
    masked = jnp.where(one1, -jnp.inf, logits)
    i2 = jnp.argmax(masked, axis=-1)[:, None]
    one2 = iota == i2
    v2 = jnp.max(masked, axis=-1, keepdims=True)

    t = jnp.exp(v2 - v1)
    denom = 1.0 + t
    p1 = 1.0 / denom
    p2 = t / denom
    gates = jnp.where(one1, p1, 0.0) + jnp.where(one2, p2, 0.0)

    acc = jnp.dot(gates, b_ref[...], preferred_element_type=jnp.float32)
    xb_bf = xb.astype(jnp.bfloat16)
    for e in range(E):
        ye = jax.lax.dot_general(xb_bf, wt_ref[e], (((1,), (1,)), ((), ())),
                                 preferred_element_type=jnp.float32)
        acc = acc + gates[:, e][:, None] * ye
    o_ref[...] = acc


@jax.jit
def kernel(x, Wg, bg, W, b):
    x2 = x.reshape(S, D)
    WgT = Wg.T
    Wt = W.astype(jnp.bfloat16)
    bg2 = bg.reshape(1, E)

    out = pl.pallas_call(
        _moe_block,
        grid=(S // BS,),
        in_specs=[
            pl.BlockSpec((BS, D), lambda i: (i, 0)),
            pl.BlockSpec((D, E), lambda i: (0, 0)),
            pl.BlockSpec((1, E), lambda i: (0, 0)),
            pl.BlockSpec((E, D, D), lambda i: (0, 0, 0)),
            pl.BlockSpec((E, D), lambda i: (0, 0)),
        ],
        out_specs=pl.BlockSpec((BS, D), lambda i: (i, 0)),
        out_shape=jax.ShapeDtypeStruct((S, D), jnp.float32),
    )(x2, WgT, bg2, Wt, b)
    return out.reshape(1, S, D)


# speedup vs baseline: 6.9767x; 1.2206x over previous
import jax
import jax.numpy as jnp
from jax.experimental import pallas as pl
from jax.experimental.pallas import tpu as pltpu


S, D, E = 2048, 768, 8
BS = 512


def _moe_block(x_ref, wgt_ref, bg_ref, wt_ref, b_ref, o_ref):
    xb = x_ref[...]
    logits = jnp.dot(xb, wgt_ref[...], preferred_element_type=jnp.float32)
    logits = logits + bg_ref[...]

    iota = jax.lax.broadcasted_iota(jnp.int32, (BS, E), 1)
    i1 = jnp.argmax(logits, axis=-1)[:, None]
    one1 = iota == i1
    v1 = jnp.max(logits, axis=-1, keepdims=True)
    masked = jnp.where(one1, -jnp.inf, logits)
    i2 = jnp.argmax(masked, axis=-1)[:, None]
    one2 = iota == i2
    v2 = jnp.max(masked, axis=-1, keepdims=True)

    t = jnp.exp(v2 - v1)
    denom = 1.0 + t
    p1 = 1.0 / denom
    p2 = t / denom
    gates = jnp.where(one1, p1, 0.0) + jnp.where(one2, p2, 0.0)

    acc = jnp.dot(gates, b_ref[...], preferred_element_type=jnp.float32)
    xb_bf = xb.astype(jnp.bfloat16)
    for e in range(E):
        ye = jax.lax.dot_general(xb_bf, wt_ref[e].astype(jnp.bfloat16),
                                 (((1,), (1,)), ((), ())),
                                 preferred_element_type=jnp.float32)
        acc = acc + gates[:, e][:, None] * ye
    o_ref[...] = acc


@jax.jit
def kernel(x, Wg, bg, W, b):
    x2 = x.reshape(S, D)
    WgT = Wg.T
    bg2 = bg.reshape(1, E)

    out = pl.pallas_call(
        _moe_block,
        grid=(S // BS,),
        in_specs=[
            pl.BlockSpec((BS, D), lambda i: (i, 0)),
            pl.BlockSpec((D, E), lambda i: (0, 0)),
            pl.BlockSpec((1, E), lambda i: (0, 0)),
            pl.BlockSpec((E, D, D), lambda i: (0, 0, 0)),
            pl.BlockSpec((E, D), lambda i: (0, 0)),
        ],
        out_specs=pl.BlockSpec((BS, D), lambda i: (i, 0)),
        out_shape=jax.ShapeDtypeStruct((S, D), jnp.float32),
    )(x2, WgT, bg2, W, b)
    return out.reshape(1, S, D)


# speedup vs baseline: 7.0323x; 1.0080x over previous
import jax
import jax.numpy as jnp
from jax.experimental import pallas as pl
from jax.experimental.pallas import tpu as pltpu


S, D, E = 2048, 768, 8
BS = 1024


def _moe_block(x_ref, wgt_ref, bg_ref, wt_ref, b_ref, o_ref):
    xb = x_ref[...]
    logits = jnp.dot(xb, wgt_ref[...], preferred_element_type=jnp.float32)
    logits = logits + bg_ref[...]

    iota = jax.lax.broadcasted_iota(jnp.int32, (BS, E), 1)
    i1 = jnp.argmax(logits, axis=-1)[:, None]
    one1 = iota == i1
    v1 = jnp.max(logits, axis=-1, keepdims=True)
    masked = jnp.where(one1, -jnp.inf, logits)
    i2 = jnp.argmax(masked, axis=-1)[:, None]
    one2 = iota == i2
    v2 = jnp.max(masked, axis=-1, keepdims=True)

    t = jnp.exp(v2 - v1)
    denom = 1.0 + t
    p1 = 1.0 / denom
    p2 = t / denom
    gates = jnp.where(one1, p1, 0.0) + jnp.where(one2, p2, 0.0)

    acc = jnp.dot(gates, b_ref[...], preferred_element_type=jnp.float32)
    xb_bf = xb.astype(jnp.bfloat16)
    for e in range(E):
        ye = jax.lax.dot_general(xb_bf, wt_ref[e].astype(jnp.bfloat16),
                                 (((1,), (1,)), ((), ())),
                                 preferred_element_type=jnp.float32)
        acc = acc + gates[:, e][:, None] * ye
    o_ref[...] = acc


@jax.jit
def kernel(x, Wg, bg, W, b):
    x2 = x.reshape(S, D)
    WgT = Wg.T
    bg2 = bg.reshape(1, E)

    out = pl.pallas_call(
        _moe_block,
        grid=(S // BS,),
        in_specs=[
            pl.BlockSpec((BS, D), lambda i: (i, 0)),
            pl.BlockSpec((D, E), lambda i: (0, 0)),
            pl.BlockSpec((1, E), lambda i: (0, 0)),
            pl.BlockSpec((E, D, D), lambda i: (0, 0, 0)),
            pl.BlockSpec((E, D), lambda i: (0, 0)),
        ],
        out_specs=pl.BlockSpec((BS, D), lambda i: (i, 0)),
        out_shape=jax.ShapeDtypeStruct((S, D), jnp.float32),
    )(x2, WgT, bg2, W, b)
    return out.reshape(1, S, D)
